# Initial kernel scaffold; baseline (speedup 1.0000x reference)
import jax
import jax.numpy as jnp
from jax.experimental import pallas as pl


def kernel(x, edge_index, W1, b1, W2, b2):
    raise NotImplementedError("write your pallas kernel here")


# speedup vs baseline: 29.5672x; 29.5672x over previous
import functools

import jax
import jax.numpy as jnp
from jax import lax
from jax.experimental import pallas as pl
from jax.experimental.pallas import tpu as pltpu
from jax.experimental.pallas import tpu_sc as plsc

N = 100000
F = 32
FH = 16
H = 64
E = 1600000

NC, NS, L = 2, 16, 16
NPAD = 102400
JUNK = 102392
CH = 128
EP = 1703936
ET = EP // NS
NCHT = ET // CH
ECH = EP // CH
RPT = NPAD // NS
BLKA = 8
BLKB = 8

def _deg_body(dst2d, zeros1, out, sp_deg, idx, ones, _):
    c = lax.axis_index("c")
    s = lax.axis_index("s")

    @pl.when(s == 0)
    def _():
        pltpu.sync_copy(zeros1, sp_deg)

    for k in range(CH // L):
        ones[pl.ds(k * L, L)] = jnp.full((L,), 1.0, jnp.float32)
    plsc.subcore_barrier()

    half = ECH // 2
    per_tile = half // NS
    base = c * half + s * per_tile
    nblk = per_tile // BLKA

    def blk(b, carry):
        row0 = base + b * BLKA
        pltpu.sync_copy(dst2d.at[pl.ds(row0, BLKA)], idx)
        for j in range(BLKA):
            pltpu.sync_copy(ones, sp_deg.at[idx.at[j]], add=True)
        return carry

    lax.fori_loop(0, nblk, blk, 0)
    plsc.subcore_barrier()
    pltpu.sync_copy(sp_deg.at[pl.ds(s * RPT, RPT)],
                    out.at[pl.ds(c * NPAD + s * RPT, RPT)])


@functools.lru_cache(maxsize=None)
def _deg_call():
    mesh = plsc.VectorSubcoreMesh(core_axis_name="c", subcore_axis_name="s",
                                  num_cores=NC, num_subcores=NS)
    return pl.kernel(
        _deg_body,
        out_type=jax.ShapeDtypeStruct((NC * NPAD,), jnp.float32),
        mesh=mesh,
        compiler_params=pltpu.CompilerParams(use_tc_tiling_on_sc=False),
        scratch_types=[
            pltpu.VMEM_SHARED((NPAD,), jnp.float32),
            pltpu.VMEM((BLKA, CH), jnp.int32),
            pltpu.VMEM((CH,), jnp.float32),
            pltpu.SemaphoreType.DMA,
        ],
    )


def _agg_body(srcadj, dst2d, p_all, dinv_h, z16, z2n, agg_out, cs_out,
              sp_agg, sp_cs, sp_dinv, sidx, didx, rowbuf, csbuf, gsem):
    c = lax.axis_index("c")
    s = lax.axis_index("s")

    @pl.when(s == 0)
    def _():
        pltpu.sync_copy(z16, sp_agg)
        pltpu.sync_copy(z2n, sp_cs)

    @pl.when(s == 1)
    def _():
        pltpu.sync_copy(dinv_h, sp_dinv)

    plsc.subcore_barrier()

    nblk = NCHT // BLKB
    halfblk = nblk // 2

    def blk(b, carry):
        row0 = s * NCHT + b * BLKB
        pltpu.sync_copy(srcadj.at[pl.ds(c * ECH + row0, BLKB)], sidx)
        pltpu.sync_copy(dst2d.at[pl.ds(row0, BLKB)], didx)
        for j in range(BLKB):
            pltpu.async_copy(p_all.at[sidx.at[j]], rowbuf, gsem).wait()
            pltpu.sync_copy(rowbuf, sp_agg.at[didx.at[j]], add=True)

        do_cs = jnp.logical_xor(b < halfblk, c != 0)

        @pl.when(do_cs)
        def _():
            for j in range(BLKB):
                pltpu.async_copy(sp_dinv.at[didx.at[j]], csbuf, gsem).wait()
                pltpu.sync_copy(csbuf, sp_cs.at[sidx.at[j]], add=True)

        return carry

    lax.fori_loop(0, nblk, blk, 0)
    plsc.subcore_barrier()
    pltpu.sync_copy(sp_agg.at[pl.ds(s * RPT, RPT)],
                    agg_out.at[pl.ds(c * NPAD + s * RPT, RPT)])
    pltpu.sync_copy(sp_cs.at[pl.ds(c * NPAD + s * RPT, RPT)],
                    cs_out.at[pl.ds(c * NPAD + s * RPT, RPT)])


@functools.lru_cache(maxsize=None)
def _agg_call():
    mesh = plsc.VectorSubcoreMesh(core_axis_name="c", subcore_axis_name="s",
                                  num_cores=NC, num_subcores=NS)
    return pl.kernel(
        _agg_body,
        out_type=(jax.ShapeDtypeStruct((NC * NPAD, FH), jnp.float32),
                  jax.ShapeDtypeStruct((NC * NPAD,), jnp.float32)),
        mesh=mesh,
        compiler_params=pltpu.CompilerParams(use_tc_tiling_on_sc=False),
        scratch_types=[
            pltpu.VMEM_SHARED((NPAD, FH), jnp.float32),
            pltpu.VMEM_SHARED((NC * NPAD,), jnp.float32),
            pltpu.VMEM_SHARED((NPAD,), jnp.float32),
            pltpu.VMEM((BLKB, CH), jnp.int32),
            pltpu.VMEM((BLKB, CH), jnp.int32),
            pltpu.VMEM((CH, FH), jnp.float32),
            pltpu.VMEM((CH,), jnp.float32),
            pltpu.SemaphoreType.DMA,
        ],
    )


BC = 4096


def _final_body(aggA, aggB, dinv1, cw1, W1r, b1r, W2r, b2r, out, acc):
    i = pl.program_id(0)

    @pl.when(i == 0)
    def _():
        acc[...] = jnp.zeros_like(acc)

    t = jnp.concatenate([aggA[...], aggB[...]], axis=1) * dinv1[...]
    h = jnp.dot(t, W1r[...], preferred_element_type=jnp.float32)
    h = jnp.maximum(h + b1r[...], 0.0)
    acc[...] += jnp.sum(h * cw1[...], axis=0, keepdims=True)

    @pl.when(i == pl.num_programs(0) - 1)
    def _():
        out[...] = (jnp.dot(acc[...] * (1.0 / N), W2r[...],
                            preferred_element_type=jnp.float32) + b2r[...])


def _final_call(agg, dinv1, cw1, W1, b1r, W2, b2r):
    grid = NPAD // BC
    return pl.pallas_call(
        _final_body,
        grid=(grid,),
        in_specs=[
            pl.BlockSpec((BC, FH), lambda i: (i, 0)),
            pl.BlockSpec((BC, FH), lambda i: (i + NPAD // BC, 0)),
            pl.BlockSpec((BC, 1), lambda i: (i, 0)),
            pl.BlockSpec((BC, 1), lambda i: (i, 0)),
            pl.BlockSpec((F, H), lambda i: (0, 0)),
            pl.BlockSpec((1, H), lambda i: (0, 0)),
            pl.BlockSpec((H, H), lambda i: (0, 0)),
            pl.BlockSpec((1, H), lambda i: (0, 0)),
        ],
        out_specs=pl.BlockSpec((1, H), lambda i: (0, 0)),
        out_shape=jax.ShapeDtypeStruct((1, H), jnp.float32),
        scratch_shapes=[pltpu.VMEM((1, H), jnp.float32)],
    )(agg, agg, dinv1, cw1, W1, b1r, W2, b2r)


def kernel(x, edge_index, W1, b1, W2, b2):
    ei = edge_index.astype(jnp.int32)
    src, dst = ei[0], ei[1]
    ar = jnp.arange(N, dtype=jnp.int32)

    npd = EP - (E + N)
    padsrc = (jnp.arange(npd, dtype=jnp.int32) * 997) % N
    paddst = JUNK + (jnp.arange(npd, dtype=jnp.int32) % 8)
    srcf = jnp.concatenate([src, ar, padsrc])
    dstf = jnp.concatenate([dst, ar, paddst])
    src2d = srcf.reshape(ECH, CH)
    srcadj = jnp.concatenate([src2d, src2d + NPAD], axis=0)
    dst2d = dstf.reshape(ECH, CH)

    z1 = jnp.zeros((NPAD,), jnp.float32)
    z16 = jnp.zeros((NPAD, FH), jnp.float32)
    z2n = jnp.zeros((NC * NPAD,), jnp.float32)

    degp = _deg_call()(dst2d, z1)
    deg = degp[:NPAD] + degp[NPAD:]
    node_mask = jnp.arange(NPAD) < N
    dinv = jnp.where(node_mask, lax.rsqrt(jnp.maximum(deg, 1.0)), 0.0)

    p = x * dinv[:N, None]
    pA = jnp.pad(p[:, :FH], ((0, NPAD - N), (0, 0)))
    pB = jnp.pad(p[:, FH:], ((0, NPAD - N), (0, 0)))
    p_all = jnp.concatenate([pA, pB], axis=0)

    agg, csp = _agg_call()(srcadj, dst2d, p_all, dinv, z16, z2n)
    csum = csp[:NPAD] + csp[NPAD:]
    cw = jnp.where(node_mask, dinv * csum, 0.0)

    return _final_call(agg, dinv.reshape(NPAD, 1), cw.reshape(NPAD, 1),
                       W1, b1.reshape(1, H), W2, b2.reshape(1, H))


# speedup vs baseline: 44.0036x; 1.4883x over previous
import functools

import jax
import jax.numpy as jnp
from jax import lax
from jax.experimental import pallas as pl
from jax.experimental.pallas import tpu as pltpu
from jax.experimental.pallas import tpu_sc as plsc

N = 100000
F = 32
FH = 16
H = 64
E = 1600000

NC, NS, L = 2, 16, 16
NPAD = 100352
JUNK = 100344
CH = 128
EP = 1703936
ET = EP // NS
NCHT = ET // CH
ECH = EP // CH
RPT = NPAD // NS
BLKA = 8
BLKB = 4

def _deg_body(dst2d, zeros1, out, sp_deg, idx, ones, _):
    c = lax.axis_index("c")
    s = lax.axis_index("s")

    @pl.when(s == 0)
    def _():
        pltpu.sync_copy(zeros1, sp_deg)

    for k in range(CH // L):
        ones[pl.ds(k * L, L)] = jnp.full((L,), 1.0, jnp.float32)
    plsc.subcore_barrier()

    half = ECH // 2
    per_tile = half // NS
    base = c * half + s * per_tile
    nblk = per_tile // BLKA

    def blk(b, carry):
        row0 = base + b * BLKA
        pltpu.sync_copy(dst2d.at[pl.ds(row0, BLKA)], idx)
        for j in range(BLKA):
            pltpu.sync_copy(ones, sp_deg.at[idx.at[j]], add=True)
        return carry

    lax.fori_loop(0, nblk, blk, 0)
    plsc.subcore_barrier()
    pltpu.sync_copy(sp_deg.at[pl.ds(s * RPT, RPT)],
                    out.at[pl.ds(c * NPAD + s * RPT, RPT)])


@functools.lru_cache(maxsize=None)
def _deg_call():
    mesh = plsc.VectorSubcoreMesh(core_axis_name="c", subcore_axis_name="s",
                                  num_cores=NC, num_subcores=NS)
    return pl.kernel(
        _deg_body,
        out_type=jax.ShapeDtypeStruct((NC * NPAD,), jnp.float32),
        mesh=mesh,
        compiler_params=pltpu.CompilerParams(use_tc_tiling_on_sc=False),
        scratch_types=[
            pltpu.VMEM_SHARED((NPAD,), jnp.float32),
            pltpu.VMEM((BLKA, CH), jnp.int32),
            pltpu.VMEM((CH,), jnp.float32),
            pltpu.SemaphoreType.DMA,
        ],
    )


def _agg_body(src2d, dst2d, p0, p1, dinv_h, z16, z1, agg_out, cs_out,
              sp_agg, sp_cs, sp_dinv, sidx, didx, rowbuf, csbuf,
              gsem, ssem, isem, cgsem, cssem):
    c = lax.axis_index("c")
    s = lax.axis_index("s")

    @pl.when(s == 0)
    def _():
        pltpu.sync_copy(z16, sp_agg)
        pltpu.sync_copy(z1, sp_cs)

    @pl.when(s == 1)
    def _():
        pltpu.sync_copy(dinv_h, sp_dinv)

    plsc.subcore_barrier()

    nblk = NCHT // BLKB
    halfblk = nblk // 2

    def idx_fetch(b, slot):
        row0 = s * NCHT + b * BLKB
        pltpu.async_copy(src2d.at[pl.ds(row0, BLKB)], sidx.at[slot], isem)
        pltpu.async_copy(dst2d.at[pl.ds(row0, BLKB)], didx.at[slot], isem)

    def cs_active(b):
        return jnp.logical_xor(b < halfblk, c != 0)

    def drain_agg(slot4):
        for j in range(BLKB):
            pltpu.make_async_copy(rowbuf.at[j],
                                  sp_agg.at[didx.at[slot4, j]], ssem).wait()

    def drain_cs(slot4):
        for j in range(BLKB):
            pltpu.make_async_copy(csbuf.at[j],
                                  sp_cs.at[sidx.at[slot4, j]], cssem).wait()

    idx_fetch(0, 0)

    def blk(b, carry):
        p4 = lax.rem(b, 4)
        pltpu.make_async_copy(src2d.at[pl.ds(0, BLKB)],
                              sidx.at[p4], isem).wait()
        pltpu.make_async_copy(dst2d.at[pl.ds(0, BLKB)],
                              didx.at[p4], isem).wait()

        @pl.when(b + 1 < nblk)
        def _():
            idx_fetch(b + 1, lax.rem(b + 1, 4))

        @pl.when(b >= 1)
        def _():
            drain_agg(p4)

        @pl.when(jnp.logical_and(b >= 1, cs_active(b - 1)))
        def _():
            drain_cs(p4)

        def p_pipe(p_half):
            gd = [pltpu.async_copy(p_half.at[sidx.at[p4, j]],
                                   rowbuf.at[j], gsem)
                  for j in range(BLKB)]
            for j in range(BLKB):
                gd[j].wait()
                pltpu.async_copy(rowbuf.at[j],
                                 sp_agg.at[didx.at[p4, j]], ssem, add=True)

        @pl.when(c == 0)
        def _():
            p_pipe(p0)

        @pl.when(c == 1)
        def _():
            p_pipe(p1)

        @pl.when(cs_active(b))
        def _():
            cg = [pltpu.async_copy(sp_dinv.at[didx.at[p4, j]],
                                   csbuf.at[j], cgsem)
                  for j in range(BLKB)]
            for j in range(BLKB):
                cg[j].wait()
                pltpu.async_copy(csbuf.at[j],
                                 sp_cs.at[sidx.at[p4, j]], cssem, add=True)

        return carry

    lax.fori_loop(0, nblk, blk, 0)

    drain_agg((nblk - 1) % 4)

    @pl.when(c == 1)
    def _():
        drain_cs((nblk - 1) % 4)

    plsc.subcore_barrier()
    pltpu.sync_copy(sp_agg.at[pl.ds(s * RPT, RPT)],
                    agg_out.at[pl.ds(c * NPAD + s * RPT, RPT)])
    pltpu.sync_copy(sp_cs.at[pl.ds(s * RPT, RPT)],
                    cs_out.at[pl.ds(c * NPAD + s * RPT, RPT)])


@functools.lru_cache(maxsize=None)
def _agg_call():
    mesh = plsc.VectorSubcoreMesh(core_axis_name="c", subcore_axis_name="s",
                                  num_cores=NC, num_subcores=NS)
    return pl.kernel(
        _agg_body,
        out_type=(jax.ShapeDtypeStruct((NC * NPAD, FH), jnp.float32),
                  jax.ShapeDtypeStruct((NC * NPAD,), jnp.float32)),
        mesh=mesh,
        compiler_params=pltpu.CompilerParams(use_tc_tiling_on_sc=False),
        scratch_types=[
            pltpu.VMEM_SHARED((NPAD, FH), jnp.float32),
            pltpu.VMEM_SHARED((NPAD,), jnp.float32),
            pltpu.VMEM_SHARED((NPAD,), jnp.float32),
            pltpu.VMEM((4, BLKB, CH), jnp.int32),
            pltpu.VMEM((4, BLKB, CH), jnp.int32),
            pltpu.VMEM((BLKB, CH, FH), jnp.float32),
            pltpu.VMEM((BLKB, CH), jnp.float32),
            pltpu.SemaphoreType.DMA,
            pltpu.SemaphoreType.DMA,
            pltpu.SemaphoreType.DMA,
            pltpu.SemaphoreType.DMA,
            pltpu.SemaphoreType.DMA,
        ],
    )


BC = 2048


def _final_body(aggA, aggB, dinv1, cw1, W1r, b1r, W2r, b2r, out, acc):
    i = pl.program_id(0)

    @pl.when(i == 0)
    def _():
        acc[...] = jnp.zeros_like(acc)

    t = jnp.concatenate([aggA[...], aggB[...]], axis=1) * dinv1[...]
    h = jnp.dot(t, W1r[...], preferred_element_type=jnp.float32)
    h = jnp.maximum(h + b1r[...], 0.0)
    acc[...] += jnp.sum(h * cw1[...], axis=0, keepdims=True)

    @pl.when(i == pl.num_programs(0) - 1)
    def _():
        out[...] = (jnp.dot(acc[...] * (1.0 / N), W2r[...],
                            preferred_element_type=jnp.float32) + b2r[...])


def _final_call(agg, dinv1, cw1, W1, b1r, W2, b2r):
    grid = NPAD // BC
    return pl.pallas_call(
        _final_body,
        grid=(grid,),
        in_specs=[
            pl.BlockSpec((BC, FH), lambda i: (i, 0)),
            pl.BlockSpec((BC, FH), lambda i: (i + NPAD // BC, 0)),
            pl.BlockSpec((BC, 1), lambda i: (i, 0)),
            pl.BlockSpec((BC, 1), lambda i: (i, 0)),
            pl.BlockSpec((F, H), lambda i: (0, 0)),
            pl.BlockSpec((1, H), lambda i: (0, 0)),
            pl.BlockSpec((H, H), lambda i: (0, 0)),
            pl.BlockSpec((1, H), lambda i: (0, 0)),
        ],
        out_specs=pl.BlockSpec((1, H), lambda i: (0, 0)),
        out_shape=jax.ShapeDtypeStruct((1, H), jnp.float32),
        scratch_shapes=[pltpu.VMEM((1, H), jnp.float32)],
    )(agg, agg, dinv1, cw1, W1, b1r, W2, b2r)


def kernel(x, edge_index, W1, b1, W2, b2):
    ei = edge_index.astype(jnp.int32)
    src, dst = ei[0], ei[1]
    ar = jnp.arange(N, dtype=jnp.int32)

    npd = EP - (E + N)
    padsrc = (jnp.arange(npd, dtype=jnp.int32) * 997) % N
    paddst = JUNK + (jnp.arange(npd, dtype=jnp.int32) % 8)
    srcf = jnp.concatenate([src, ar, padsrc])
    dstf = jnp.concatenate([dst, ar, paddst])
    src2d = srcf.reshape(ECH, CH)
    dst2d = dstf.reshape(ECH, CH)

    z1 = jnp.zeros((NPAD,), jnp.float32)
    z16 = jnp.zeros((NPAD, FH), jnp.float32)

    degp = _deg_call()(dst2d, z1)
    deg = degp[:NPAD] + degp[NPAD:]
    node_mask = jnp.arange(NPAD) < N
    dinv = jnp.where(node_mask, lax.rsqrt(jnp.maximum(deg, 1.0)), 0.0)

    p = x * dinv[:N, None]
    pA = jnp.pad(p[:, :FH], ((0, NPAD - N), (0, 0)))
    pB = jnp.pad(p[:, FH:], ((0, NPAD - N), (0, 0)))

    agg, csp = _agg_call()(src2d, dst2d, pA, pB, dinv, z16, z1)
    csum = csp[:NPAD] + csp[NPAD:]
    cw = jnp.where(node_mask, dinv * csum, 0.0)

    return _final_call(agg, dinv.reshape(NPAD, 1), cw.reshape(NPAD, 1),
                       W1, b1.reshape(1, H), W2, b2.reshape(1, H))


# speedup vs baseline: 54.3411x; 1.2349x over previous
import functools

import jax
import jax.numpy as jnp
from jax import lax
from jax.experimental import pallas as pl
from jax.experimental.pallas import tpu as pltpu
from jax.experimental.pallas import tpu_sc as plsc

N = 100000
F = 32
FH = 16
H = 64
E = 1600000

NC, NS, L = 2, 16, 16
NPAD = 100352
JUNK = 100344
CH = 128
EP = 1703936
ET = EP // NS
NCHT = ET // CH
ECH = EP // CH
RPT = NPAD // NS
BLKA = 8
BLKB = 4

def _deg_body(dst2d, zeros1, out, sp_deg, idx, ones, _):
    c = lax.axis_index("c")
    s = lax.axis_index("s")

    @pl.when(s == 0)
    def _():
        pltpu.sync_copy(zeros1, sp_deg)

    for k in range(CH // L):
        ones[pl.ds(k * L, L)] = jnp.full((L,), 1.0, jnp.float32)
    plsc.subcore_barrier()

    half = ECH // 2
    per_tile = half // NS
    base = c * half + s * per_tile
    nblk = per_tile // BLKA

    def blk(b, carry):
        row0 = base + b * BLKA
        pltpu.sync_copy(dst2d.at[pl.ds(row0, BLKA)], idx)
        for j in range(BLKA):
            pltpu.sync_copy(ones, sp_deg.at[idx.at[j]], add=True)
        return carry

    lax.fori_loop(0, nblk, blk, 0)
    plsc.subcore_barrier()
    pltpu.sync_copy(sp_deg.at[pl.ds(s * RPT, RPT)],
                    out.at[pl.ds(c * NPAD + s * RPT, RPT)])


@functools.lru_cache(maxsize=None)
def _deg_call():
    mesh = plsc.VectorSubcoreMesh(core_axis_name="c", subcore_axis_name="s",
                                  num_cores=NC, num_subcores=NS)
    return pl.kernel(
        _deg_body,
        out_type=jax.ShapeDtypeStruct((NC * NPAD,), jnp.float32),
        mesh=mesh,
        compiler_params=pltpu.CompilerParams(use_tc_tiling_on_sc=False),
        scratch_types=[
            pltpu.VMEM_SHARED((NPAD,), jnp.float32),
            pltpu.VMEM((BLKA, CH), jnp.int32),
            pltpu.VMEM((CH,), jnp.float32),
            pltpu.SemaphoreType.DMA,
        ],
    )


def _agg_body(src2d, dst2d, p0, p1, dinv_h, z16, z1, agg_out, cs_out,
              sp_agg, sp_cs, sp_dinv, sidx, didx, rowbuf, csbuf,
              gsem, ssem, isem, cgsem, cssem):
    c = lax.axis_index("c")
    s = lax.axis_index("s")

    @pl.when(s == 0)
    def _():
        pltpu.sync_copy(z16, sp_agg)
        pltpu.sync_copy(z1, sp_cs)

    @pl.when(s == 1)
    def _():
        pltpu.sync_copy(dinv_h, sp_dinv)

    plsc.subcore_barrier()

    nblk = NCHT // BLKB
    halfblk = nblk // 2

    def idx_fetch(b, slot):
        row0 = s * NCHT + b * BLKB
        pltpu.async_copy(src2d.at[pl.ds(row0, BLKB)], sidx.at[slot], isem)
        pltpu.async_copy(dst2d.at[pl.ds(row0, BLKB)], didx.at[slot], isem)

    def cs_active(b):
        return jnp.logical_xor(b < halfblk, c != 0)

    def drain_agg(slot4):
        for j in range(BLKB):
            pltpu.make_async_copy(rowbuf.at[j],
                                  sp_agg.at[didx.at[slot4, j]], ssem).wait()

    def drain_cs(slot4):
        for j in range(BLKB):
            pltpu.make_async_copy(csbuf.at[j],
                                  sp_cs.at[sidx.at[slot4, j]], cssem).wait()

    idx_fetch(0, 0)

    def blk(b, carry):
        p4 = lax.rem(b, 4)
        pltpu.make_async_copy(src2d.at[pl.ds(0, BLKB)],
                              sidx.at[p4], isem).wait()
        pltpu.make_async_copy(dst2d.at[pl.ds(0, BLKB)],
                              didx.at[p4], isem).wait()

        @pl.when(b + 1 < nblk)
        def _():
            idx_fetch(b + 1, lax.rem(b + 1, 4))

        @pl.when(b >= 1)
        def _():
            drain_agg(p4)

        @pl.when(jnp.logical_and(b >= 1, cs_active(b - 1)))
        def _():
            drain_cs(p4)

        @pl.when(cs_active(b))
        def _():
            for j in range(BLKB):
                pltpu.async_copy(sp_dinv.at[didx.at[p4, j]],
                                 csbuf.at[j], cgsem)

        def p_pipe(p_half):
            gd = [pltpu.async_copy(p_half.at[sidx.at[p4, j]],
                                   rowbuf.at[j], gsem)
                  for j in range(BLKB)]
            for j in range(BLKB):
                gd[j].wait()
                pltpu.async_copy(rowbuf.at[j],
                                 sp_agg.at[didx.at[p4, j]], ssem, add=True)

        @pl.when(c == 0)
        def _():
            p_pipe(p0)

        @pl.when(c == 1)
        def _():
            p_pipe(p1)

        @pl.when(cs_active(b))
        def _():
            for j in range(BLKB):
                pltpu.make_async_copy(sp_dinv.at[didx.at[p4, j]],
                                      csbuf.at[j], cgsem).wait()
                pltpu.async_copy(csbuf.at[j],
                                 sp_cs.at[sidx.at[p4, j]], cssem, add=True)

        return carry

    lax.fori_loop(0, nblk, blk, 0)

    drain_agg((nblk - 1) % 4)

    @pl.when(c == 1)
    def _():
        drain_cs((nblk - 1) % 4)

    plsc.subcore_barrier()
    pltpu.sync_copy(sp_agg.at[pl.ds(s * RPT, RPT)],
                    agg_out.at[pl.ds(c * NPAD + s * RPT, RPT)])
    pltpu.sync_copy(sp_cs.at[pl.ds(s * RPT, RPT)],
                    cs_out.at[pl.ds(c * NPAD + s * RPT, RPT)])


@functools.lru_cache(maxsize=None)
def _agg_call():
    mesh = plsc.VectorSubcoreMesh(core_axis_name="c", subcore_axis_name="s",
                                  num_cores=NC, num_subcores=NS)
    return pl.kernel(
        _agg_body,
        out_type=(jax.ShapeDtypeStruct((NC * NPAD, FH), jnp.float32),
                  jax.ShapeDtypeStruct((NC * NPAD,), jnp.float32)),
        mesh=mesh,
        compiler_params=pltpu.CompilerParams(use_tc_tiling_on_sc=False),
        scratch_types=[
            pltpu.VMEM_SHARED((NPAD, FH), jnp.float32),
            pltpu.VMEM_SHARED((NPAD,), jnp.float32),
            pltpu.VMEM_SHARED((NPAD,), jnp.float32),
            pltpu.VMEM((4, BLKB, CH), jnp.int32),
            pltpu.VMEM((4, BLKB, CH), jnp.int32),
            pltpu.VMEM((BLKB, CH, FH), jnp.float32),
            pltpu.VMEM((BLKB, CH), jnp.float32),
            pltpu.SemaphoreType.DMA,
            pltpu.SemaphoreType.DMA,
            pltpu.SemaphoreType.DMA,
            pltpu.SemaphoreType.DMA,
            pltpu.SemaphoreType.DMA,
        ],
    )


BC = 2048


NBC = NPAD // BC


def _prep_body(degA, degB, xr, pA, pB, dinvo):
    i = pl.program_id(0)
    deg = degA[0] + degB[0]
    col = jax.lax.broadcasted_iota(jnp.int32, (1, BC), 1) + i * BC
    dinvrow = jnp.where(col < N, lax.rsqrt(jnp.maximum(deg, 1.0)), 0.0)
    dinvo[...] = dinvrow.reshape(1, 1, BC)
    dcol = dinvrow.reshape(BC, 1)
    xb = xr[...]
    pA[...] = xb[:, :FH] * dcol
    pB[...] = xb[:, FH:] * dcol


def _prep_call(deg2, x):
    return pl.pallas_call(
        _prep_body,
        grid=(NBC,),
        in_specs=[
            pl.BlockSpec((1, 1, BC), lambda i: (i, 0, 0)),
            pl.BlockSpec((1, 1, BC), lambda i: (i + NBC, 0, 0)),
            pl.BlockSpec((BC, F), lambda i: (i, 0)),
        ],
        out_specs=[
            pl.BlockSpec((BC, FH), lambda i: (i, 0)),
            pl.BlockSpec((BC, FH), lambda i: (i, 0)),
            pl.BlockSpec((1, 1, BC), lambda i: (i, 0, 0)),
        ],
        out_shape=[
            jax.ShapeDtypeStruct((NPAD, FH), jnp.float32),
            jax.ShapeDtypeStruct((NPAD, FH), jnp.float32),
            jax.ShapeDtypeStruct((NBC, 1, BC), jnp.float32),
        ],
    )(deg2, deg2, x)


def _final_body(aggA, aggB, dinvr, cwr, W1r, b1c, W2r, b2r, out, acc):
    i = pl.program_id(0)

    @pl.when(i == 0)
    def _():
        acc[...] = jnp.zeros_like(acc)

    t = jnp.concatenate([aggA[...], aggB[...]], axis=1)
    mT = lax.dot_general(W1r[...], t, (((0,), (1,)), ((), ())),
                         preferred_element_type=jnp.float32)
    hT = jnp.maximum(mT * dinvr[0] + b1c[...], 0.0)
    acc[...] += jnp.sum(hT * cwr[0], axis=1, keepdims=True)

    @pl.when(i == pl.num_programs(0) - 1)
    def _():
        vT = lax.dot_general(W2r[...], acc[...] * (1.0 / N),
                             (((0,), (0,)), ((), ())),
                             preferred_element_type=jnp.float32)
        out[...] = vT.reshape(1, H) + b2r[...]


def _final_call(agg, dinvr, cwr, W1, b1c, W2, b2r):
    return pl.pallas_call(
        _final_body,
        grid=(NBC,),
        in_specs=[
            pl.BlockSpec((BC, FH), lambda i: (i, 0)),
            pl.BlockSpec((BC, FH), lambda i: (i + NBC, 0)),
            pl.BlockSpec((1, 1, BC), lambda i: (i, 0, 0)),
            pl.BlockSpec((1, 1, BC), lambda i: (i, 0, 0)),
            pl.BlockSpec((F, H), lambda i: (0, 0)),
            pl.BlockSpec((H, 1), lambda i: (0, 0)),
            pl.BlockSpec((H, H), lambda i: (0, 0)),
            pl.BlockSpec((1, H), lambda i: (0, 0)),
        ],
        out_specs=pl.BlockSpec((1, H), lambda i: (0, 0)),
        out_shape=jax.ShapeDtypeStruct((1, H), jnp.float32),
        scratch_shapes=[pltpu.VMEM((H, 1), jnp.float32)],
    )(agg, agg, dinvr, cwr, W1, b1c, W2, b2r)


def kernel(x, edge_index, W1, b1, W2, b2):
    ei = edge_index.astype(jnp.int32)
    src, dst = ei[0], ei[1]
    ar = jnp.arange(N, dtype=jnp.int32)

    npd = EP - (E + N)
    padsrc = (jnp.arange(npd, dtype=jnp.int32) * 997) % N
    paddst = JUNK + (jnp.arange(npd, dtype=jnp.int32) % 8)
    srcf = jnp.concatenate([src, ar, padsrc])
    dstf = jnp.concatenate([dst, ar, paddst])
    src2d = srcf.reshape(ECH, CH)
    dst2d = dstf.reshape(ECH, CH)

    z1 = jnp.zeros((NPAD,), jnp.float32)
    z16 = jnp.zeros((NPAD, FH), jnp.float32)

    degp = _deg_call()(dst2d, z1)
    pA, pB, dinv49 = _prep_call(degp.reshape(NC * NBC, 1, BC), x)
    dinv1d = dinv49.reshape(NPAD)

    agg, csp = _agg_call()(src2d, dst2d, pA, pB, dinv1d, z16, z1)
    cs2 = csp.reshape(NC * NBC, 1, BC)
    cw49 = dinv49 * (cs2[:NBC] + cs2[NBC:])

    return _final_call(agg, dinv49, cw49,
                       W1, b1.reshape(H, 1), W2, b2.reshape(1, H))


# speedup vs baseline: 57.7976x; 1.0636x over previous
import functools

import jax
import jax.numpy as jnp
from jax import lax
from jax.experimental import pallas as pl
from jax.experimental.pallas import tpu as pltpu
from jax.experimental.pallas import tpu_sc as plsc

N = 100000
F = 32
FH = 16
H = 64
E = 1600000

NC, NS, L = 2, 16, 16
NPAD = 100352
CH = 128
ECH = E // CH
RPT = NPAD // NS
ROWT = 781
EXTRA = ECH - NS * ROWT
BLKB = 4
NBLK = ROWT // BLKB
TAILROW = NBLK * BLKB
DEGB = 5
DEGROWS = ECH // (NC * NS)
DEGX = ECH - NC * NS * DEGROWS

BC = 2048
NBC = NPAD // BC


def _deg_body(dst2d, zeros1, out, sp_deg, idx, ones, _):
    c = lax.axis_index("c")
    s = lax.axis_index("s")
    wid = c * NS + s

    @pl.when(s == 0)
    def _():
        pltpu.sync_copy(zeros1, sp_deg)

    for k in range(CH // L):
        ones[pl.ds(k * L, L)] = jnp.full((L,), 1.0, jnp.float32)
    plsc.subcore_barrier()

    base = wid * DEGROWS
    nblk = DEGROWS // DEGB

    def blk(b, carry):
        row0 = base + b * DEGB
        pltpu.sync_copy(dst2d.at[pl.ds(row0, DEGB)], idx)
        for j in range(DEGB):
            pltpu.sync_copy(ones, sp_deg.at[idx.at[j]], add=True)
        return carry

    lax.fori_loop(0, nblk, blk, 0)

    @pl.when(wid < DEGX)
    def _():
        row0 = NC * NS * DEGROWS + wid
        pltpu.sync_copy(dst2d.at[pl.ds(row0, 1)], idx.at[pl.ds(0, 1)])
        pltpu.sync_copy(ones, sp_deg.at[idx.at[0]], add=True)

    plsc.subcore_barrier()
    pltpu.sync_copy(sp_deg.at[pl.ds(s * RPT, RPT)],
                    out.at[pl.ds(c * NPAD + s * RPT, RPT)])


@functools.lru_cache(maxsize=None)
def _deg_call():
    mesh = plsc.VectorSubcoreMesh(core_axis_name="c", subcore_axis_name="s",
                                  num_cores=NC, num_subcores=NS)
    return pl.kernel(
        _deg_body,
        out_type=jax.ShapeDtypeStruct((NC * NPAD,), jnp.float32),
        mesh=mesh,
        compiler_params=pltpu.CompilerParams(use_tc_tiling_on_sc=False),
        scratch_types=[
            pltpu.VMEM_SHARED((NPAD,), jnp.float32),
            pltpu.VMEM((DEGB, CH), jnp.int32),
            pltpu.VMEM((CH,), jnp.float32),
            pltpu.SemaphoreType.DMA,
        ],
    )


def _agg_body(src2d, dst2d, p0, p1, dinv_h, z16, z1, agg_out, cs_out,
              sp_agg, sp_cs, sp_dinv, sidx, didx, rowbuf, csbuf,
              gsem, ssem, isem, cgsem, cssem):
    c = lax.axis_index("c")
    s = lax.axis_index("s")

    @pl.when(s == 0)
    def _():
        pltpu.sync_copy(z16, sp_agg)
        pltpu.sync_copy(z1, sp_cs)

    @pl.when(s == 1)
    def _():
        pltpu.sync_copy(dinv_h, sp_dinv)

    plsc.subcore_barrier()

    halfblk = (NBLK + 1) // 2

    def idx_fetch(b, slot):
        row0 = s * ROWT + b * BLKB
        pltpu.async_copy(src2d.at[pl.ds(row0, BLKB)], sidx.at[slot], isem)
        pltpu.async_copy(dst2d.at[pl.ds(row0, BLKB)], didx.at[slot], isem)

    def cs_active(b):
        return jnp.logical_xor(b < halfblk, c != 0)

    def drain_agg(slot4):
        for j in range(BLKB):
            pltpu.make_async_copy(rowbuf.at[j],
                                  sp_agg.at[didx.at[slot4, j]], ssem).wait()

    def drain_cs(slot4):
        for j in range(BLKB):
            pltpu.make_async_copy(csbuf.at[j],
                                  sp_cs.at[sidx.at[slot4, j]], cssem).wait()

    idx_fetch(0, 0)

    def blk(b, carry):
        p4 = lax.rem(b, 4)
        pltpu.make_async_copy(src2d.at[pl.ds(0, BLKB)],
                              sidx.at[p4], isem).wait()
        pltpu.make_async_copy(dst2d.at[pl.ds(0, BLKB)],
                              didx.at[p4], isem).wait()

        @pl.when(b + 1 < NBLK)
        def _():
            idx_fetch(b + 1, lax.rem(b + 1, 4))

        @pl.when(b >= 1)
        def _():
            drain_agg(p4)

        @pl.when(jnp.logical_and(b >= 1, cs_active(b - 1)))
        def _():
            drain_cs(p4)

        @pl.when(cs_active(b))
        def _():
            for j in range(BLKB):
                pltpu.async_copy(sp_dinv.at[didx.at[p4, j]],
                                 csbuf.at[j], cgsem)

        def p_pipe(p_half):
            gd = [pltpu.async_copy(p_half.at[sidx.at[p4, j]],
                                   rowbuf.at[j], gsem)
                  for j in range(BLKB)]
            for j in range(BLKB):
                gd[j].wait()
                pltpu.async_copy(rowbuf.at[j],
                                 sp_agg.at[didx.at[p4, j]], ssem, add=True)

        @pl.when(c == 0)
        def _():
            p_pipe(p0)

        @pl.when(c == 1)
        def _():
            p_pipe(p1)

        @pl.when(cs_active(b))
        def _():
            for j in range(BLKB):
                pltpu.make_async_copy(sp_dinv.at[didx.at[p4, j]],
                                      csbuf.at[j], cgsem).wait()
                pltpu.async_copy(csbuf.at[j],
                                 sp_cs.at[sidx.at[p4, j]], cssem, add=True)

        return carry

    lax.fori_loop(0, NBLK, blk, 0)

    drain_agg((NBLK - 1) % 4)

    @pl.when(c == 1)
    def _():
        drain_cs((NBLK - 1) % 4)

    def tail_chunk(row0, do_cs):
        pltpu.sync_copy(src2d.at[pl.ds(row0, 1)], sidx.at[0, pl.ds(0, 1)])
        pltpu.sync_copy(dst2d.at[pl.ds(row0, 1)], didx.at[0, pl.ds(0, 1)])

        def tp_pipe(p_half):
            pltpu.async_copy(p_half.at[sidx.at[0, 0]],
                             rowbuf.at[0], gsem).wait()
            pltpu.sync_copy(rowbuf.at[0],
                            sp_agg.at[didx.at[0, 0]], add=True)

        @pl.when(c == 0)
        def _():
            tp_pipe(p0)

        @pl.when(c == 1)
        def _():
            tp_pipe(p1)

        @pl.when(do_cs)
        def _():
            pltpu.async_copy(sp_dinv.at[didx.at[0, 0]],
                             csbuf.at[0], cgsem).wait()
            pltpu.sync_copy(csbuf.at[0],
                            sp_cs.at[sidx.at[0, 0]], add=True)

    tail_chunk(s * ROWT + TAILROW, c == 0)

    @pl.when(s < EXTRA)
    def _():
        tail_chunk(NS * ROWT + s, c == 1)

    plsc.subcore_barrier()
    pltpu.sync_copy(sp_agg.at[pl.ds(s * RPT, RPT)],
                    agg_out.at[pl.ds(c * NPAD + s * RPT, RPT)])
    pltpu.sync_copy(sp_cs.at[pl.ds(s * RPT, RPT)],
                    cs_out.at[pl.ds(c * NPAD + s * RPT, RPT)])


@functools.lru_cache(maxsize=None)
def _agg_call():
    mesh = plsc.VectorSubcoreMesh(core_axis_name="c", subcore_axis_name="s",
                                  num_cores=NC, num_subcores=NS)
    return pl.kernel(
        _agg_body,
        out_type=(jax.ShapeDtypeStruct((NC * NPAD, FH), jnp.float32),
                  jax.ShapeDtypeStruct((NC * NPAD,), jnp.float32)),
        mesh=mesh,
        compiler_params=pltpu.CompilerParams(use_tc_tiling_on_sc=False),
        scratch_types=[
            pltpu.VMEM_SHARED((NPAD, FH), jnp.float32),
            pltpu.VMEM_SHARED((NPAD,), jnp.float32),
            pltpu.VMEM_SHARED((NPAD,), jnp.float32),
            pltpu.VMEM((4, BLKB, CH), jnp.int32),
            pltpu.VMEM((4, BLKB, CH), jnp.int32),
            pltpu.VMEM((BLKB, CH, FH), jnp.float32),
            pltpu.VMEM((BLKB, CH), jnp.float32),
            pltpu.SemaphoreType.DMA,
            pltpu.SemaphoreType.DMA,
            pltpu.SemaphoreType.DMA,
            pltpu.SemaphoreType.DMA,
            pltpu.SemaphoreType.DMA,
        ],
    )


def _prep_body(degA, degB, xr, pA, pB, dinvo):
    i = pl.program_id(0)
    deg = degA[...] + degB[...]
    col = lax.broadcasted_iota(jnp.int32, (BC,), 0) + i * BC
    dinv1 = jnp.where(col < N, lax.rsqrt(deg + 1.0), 0.0)
    dinvo[...] = dinv1
    dcol = dinv1.reshape(BC, 1)
    xb = xr[...]
    pA[...] = jnp.where(dcol > 0, xb[:, :FH] * dcol, 0.0)
    pB[...] = jnp.where(dcol > 0, xb[:, FH:] * dcol, 0.0)


def _prep_call(degp, x):
    return pl.pallas_call(
        _prep_body,
        grid=(NBC,),
        in_specs=[
            pl.BlockSpec((BC,), lambda i: (i,)),
            pl.BlockSpec((BC,), lambda i: (i + NBC,)),
            pl.BlockSpec((BC, F), lambda i: (i, 0)),
        ],
        out_specs=[
            pl.BlockSpec((BC, FH), lambda i: (i, 0)),
            pl.BlockSpec((BC, FH), lambda i: (i, 0)),
            pl.BlockSpec((BC,), lambda i: (i,)),
        ],
        out_shape=[
            jax.ShapeDtypeStruct((NPAD, FH), jnp.float32),
            jax.ShapeDtypeStruct((NPAD, FH), jnp.float32),
            jax.ShapeDtypeStruct((NPAD,), jnp.float32),
        ],
    )(degp, degp, x)


def _final_body(aggA, aggB, pAr, pBr, dinvb, csA, csB,
                W1r, b1c, W2r, b2r, out, acc):
    i = pl.program_id(0)

    @pl.when(i == 0)
    def _():
        acc[...] = jnp.zeros_like(acc)

    dinvrow = dinvb[...].reshape(1, BC)
    cwrow = dinvrow * (csA[...].reshape(1, BC)
                       + csB[...].reshape(1, BC) + dinvrow)
    t = jnp.concatenate([aggA[...] + pAr[...], aggB[...] + pBr[...]],
                        axis=1)
    mT = lax.dot_general(W1r[...], t, (((0,), (1,)), ((), ())),
                         preferred_element_type=jnp.float32)
    hT = jnp.maximum(mT * dinvrow + b1c[...], 0.0)
    acc[...] += jnp.sum(hT * cwrow, axis=1, keepdims=True)

    @pl.when(i == pl.num_programs(0) - 1)
    def _():
        vT = lax.dot_general(W2r[...], acc[...] * (1.0 / N),
                             (((0,), (0,)), ((), ())),
                             preferred_element_type=jnp.float32)
        out[...] = vT.reshape(1, H) + b2r[...]


def _final_call(agg, pA, pB, dinv1d, csp, W1, b1c, W2, b2r):
    return pl.pallas_call(
        _final_body,
        grid=(NBC,),
        in_specs=[
            pl.BlockSpec((BC, FH), lambda i: (i, 0)),
            pl.BlockSpec((BC, FH), lambda i: (i + NBC, 0)),
            pl.BlockSpec((BC, FH), lambda i: (i, 0)),
            pl.BlockSpec((BC, FH), lambda i: (i, 0)),
            pl.BlockSpec((BC,), lambda i: (i,)),
            pl.BlockSpec((BC,), lambda i: (i,)),
            pl.BlockSpec((BC,), lambda i: (i + NBC,)),
            pl.BlockSpec((F, H), lambda i: (0, 0)),
            pl.BlockSpec((H, 1), lambda i: (0, 0)),
            pl.BlockSpec((H, H), lambda i: (0, 0)),
            pl.BlockSpec((1, H), lambda i: (0, 0)),
        ],
        out_specs=pl.BlockSpec((1, H), lambda i: (0, 0)),
        out_shape=jax.ShapeDtypeStruct((1, H), jnp.float32),
        scratch_shapes=[pltpu.VMEM((H, 1), jnp.float32)],
    )(agg, agg, pA, pB, dinv1d, csp, csp, W1, b1c, W2, b2r)


def kernel(x, edge_index, W1, b1, W2, b2):
    ei = edge_index.astype(jnp.int32)
    src2d = ei[0].reshape(ECH, CH)
    dst2d = ei[1].reshape(ECH, CH)

    z1 = jnp.zeros((NPAD,), jnp.float32)
    z16 = jnp.zeros((NPAD, FH), jnp.float32)

    degp = _deg_call()(dst2d, z1)
    pA, pB, dinv1d = _prep_call(degp, x)
    agg, csp = _agg_call()(src2d, dst2d, pA, pB, dinv1d, z16, z1)

    return _final_call(agg, pA, pB, dinv1d, csp,
                       W1, b1.reshape(H, 1), W2, b2.reshape(1, H))


# speedup vs baseline: 60.5768x; 1.0481x over previous
import functools

import jax
import jax.numpy as jnp
from jax import lax
from jax.experimental import pallas as pl
from jax.experimental.pallas import tpu as pltpu
from jax.experimental.pallas import tpu_sc as plsc

N = 100000
F = 32
FH = 16
H = 64
E = 1600000

NC, NS, L = 2, 16, 16
NPAD = 100352
CH = 128
ECH = E // CH
RPT = NPAD // NS
ROWT = 781
EXTRA = ECH - NS * ROWT
BLKB = 4
NBLK = ROWT // BLKB
TAILROW = NBLK * BLKB
DEGB = 5
DEGROWS = ECH // (NC * NS)
DEGX = ECH - NC * NS * DEGROWS

BC = 2048
NBC = NPAD // BC


def _deg_body(dst2d, zeros1, out, sp_deg, idx, ones, isem, ssem):
    c = lax.axis_index("c")
    s = lax.axis_index("s")
    wid = c * NS + s

    @pl.when(s == 0)
    def _():
        pltpu.sync_copy(zeros1, sp_deg)

    for k in range(CH // L):
        ones[pl.ds(k * L, L)] = jnp.full((L,), 1.0, jnp.float32)
    plsc.subcore_barrier()

    base = wid * DEGROWS
    nblk = DEGROWS // DEGB

    def idx_fetch(b, slot):
        pltpu.async_copy(dst2d.at[pl.ds(base + b * DEGB, DEGB)],
                         idx.at[slot], isem)

    def drain(slot4):
        for j in range(DEGB):
            pltpu.make_async_copy(ones, sp_deg.at[idx.at[slot4, j]],
                                  ssem).wait()

    idx_fetch(0, 0)

    def blk(b, carry):
        p4 = lax.rem(b, 4)
        pltpu.make_async_copy(dst2d.at[pl.ds(0, DEGB)],
                              idx.at[p4], isem).wait()

        @pl.when(b + 1 < nblk)
        def _():
            idx_fetch(b + 1, lax.rem(b + 1, 4))

        @pl.when(b >= 1)
        def _():
            drain(p4)

        for j in range(DEGB):
            pltpu.async_copy(ones, sp_deg.at[idx.at[p4, j]], ssem, add=True)
        return carry

    lax.fori_loop(0, nblk, blk, 0)
    drain((nblk - 1) % 4)

    @pl.when(wid < DEGX)
    def _():
        row0 = NC * NS * DEGROWS + wid
        pltpu.sync_copy(dst2d.at[pl.ds(row0, 1)], idx.at[0, pl.ds(0, 1)])
        pltpu.sync_copy(ones, sp_deg.at[idx.at[0, 0]], add=True)

    plsc.subcore_barrier()
    pltpu.sync_copy(sp_deg.at[pl.ds(s * RPT, RPT)],
                    out.at[pl.ds(c * NPAD + s * RPT, RPT)])


@functools.lru_cache(maxsize=None)
def _deg_call():
    mesh = plsc.VectorSubcoreMesh(core_axis_name="c", subcore_axis_name="s",
                                  num_cores=NC, num_subcores=NS)
    return pl.kernel(
        _deg_body,
        out_type=jax.ShapeDtypeStruct((NC * NPAD,), jnp.float32),
        mesh=mesh,
        compiler_params=pltpu.CompilerParams(use_tc_tiling_on_sc=False),
        scratch_types=[
            pltpu.VMEM_SHARED((NPAD,), jnp.float32),
            pltpu.VMEM((4, DEGB, CH), jnp.int32),
            pltpu.VMEM((CH,), jnp.float32),
            pltpu.SemaphoreType.DMA,
            pltpu.SemaphoreType.DMA,
        ],
    )


def _agg_body(src2d, dst2d, p0, p1, dinv_h, z16, z1, agg_out, cs_out,
              sp_agg, sp_cs, sp_dinv, sidx, didx, rowbuf, csbuf,
              gsem, ssem, isem, cgsem, cssem):
    c = lax.axis_index("c")
    s = lax.axis_index("s")

    @pl.when(s == 0)
    def _():
        pltpu.sync_copy(z16, sp_agg)
        pltpu.sync_copy(z1, sp_cs)

    @pl.when(s == 1)
    def _():
        pltpu.sync_copy(dinv_h, sp_dinv)

    plsc.subcore_barrier()

    halfblk = (NBLK + 1) // 2

    def idx_fetch(b, slot):
        row0 = s * ROWT + b * BLKB
        pltpu.async_copy(src2d.at[pl.ds(row0, BLKB)], sidx.at[slot], isem)
        pltpu.async_copy(dst2d.at[pl.ds(row0, BLKB)], didx.at[slot], isem)

    def cs_active(b):
        return jnp.logical_xor(b < halfblk, c != 0)

    def drain_agg(slot4):
        for j in range(BLKB):
            pltpu.make_async_copy(rowbuf.at[j],
                                  sp_agg.at[didx.at[slot4, j]], ssem).wait()

    def drain_cs(slot4):
        for j in range(BLKB):
            pltpu.make_async_copy(csbuf.at[j],
                                  sp_cs.at[sidx.at[slot4, j]], cssem).wait()

    idx_fetch(0, 0)

    def blk(b, carry):
        p4 = lax.rem(b, 4)
        pltpu.make_async_copy(src2d.at[pl.ds(0, BLKB)],
                              sidx.at[p4], isem).wait()
        pltpu.make_async_copy(dst2d.at[pl.ds(0, BLKB)],
                              didx.at[p4], isem).wait()

        @pl.when(b + 1 < NBLK)
        def _():
            idx_fetch(b + 1, lax.rem(b + 1, 4))

        @pl.when(b >= 1)
        def _():
            drain_agg(p4)

        @pl.when(jnp.logical_and(b >= 1, cs_active(b - 1)))
        def _():
            drain_cs(p4)

        @pl.when(cs_active(b))
        def _():
            for j in range(BLKB):
                pltpu.async_copy(sp_dinv.at[didx.at[p4, j]],
                                 csbuf.at[j], cgsem)

        def p_pipe(p_half):
            gd = [pltpu.async_copy(p_half.at[sidx.at[p4, j]],
                                   rowbuf.at[j], gsem)
                  for j in range(BLKB)]
            for j in range(BLKB):
                gd[j].wait()
                pltpu.async_copy(rowbuf.at[j],
                                 sp_agg.at[didx.at[p4, j]], ssem, add=True)

        @pl.when(c == 0)
        def _():
            p_pipe(p0)

        @pl.when(c == 1)
        def _():
            p_pipe(p1)

        @pl.when(cs_active(b))
        def _():
            for j in range(BLKB):
                pltpu.make_async_copy(sp_dinv.at[didx.at[p4, j]],
                                      csbuf.at[j], cgsem).wait()
                pltpu.async_copy(csbuf.at[j],
                                 sp_cs.at[sidx.at[p4, j]], cssem, add=True)

        return carry

    lax.fori_loop(0, NBLK, blk, 0)

    drain_agg((NBLK - 1) % 4)

    @pl.when(c == 1)
    def _():
        drain_cs((NBLK - 1) % 4)

    def tail_chunk(row0, do_cs):
        pltpu.sync_copy(src2d.at[pl.ds(row0, 1)], sidx.at[0, pl.ds(0, 1)])
        pltpu.sync_copy(dst2d.at[pl.ds(row0, 1)], didx.at[0, pl.ds(0, 1)])

        def tp_pipe(p_half):
            pltpu.async_copy(p_half.at[sidx.at[0, 0]],
                             rowbuf.at[0], gsem).wait()
            pltpu.sync_copy(rowbuf.at[0],
                            sp_agg.at[didx.at[0, 0]], add=True)

        @pl.when(c == 0)
        def _():
            tp_pipe(p0)

        @pl.when(c == 1)
        def _():
            tp_pipe(p1)

        @pl.when(do_cs)
        def _():
            pltpu.async_copy(sp_dinv.at[didx.at[0, 0]],
                             csbuf.at[0], cgsem).wait()
            pltpu.sync_copy(csbuf.at[0],
                            sp_cs.at[sidx.at[0, 0]], add=True)

    tail_chunk(s * ROWT + TAILROW, c == 0)

    @pl.when(s < EXTRA)
    def _():
        tail_chunk(NS * ROWT + s, c == 1)

    plsc.subcore_barrier()
    pltpu.sync_copy(sp_agg.at[pl.ds(s * RPT, RPT)],
                    agg_out.at[pl.ds(c * NPAD + s * RPT, RPT)])
    pltpu.sync_copy(sp_cs.at[pl.ds(s * RPT, RPT)],
                    cs_out.at[pl.ds(c * NPAD + s * RPT, RPT)])


@functools.lru_cache(maxsize=None)
def _agg_call():
    mesh = plsc.VectorSubcoreMesh(core_axis_name="c", subcore_axis_name="s",
                                  num_cores=NC, num_subcores=NS)
    return pl.kernel(
        _agg_body,
        out_type=(jax.ShapeDtypeStruct((NC * NPAD, FH), jnp.float32),
                  jax.ShapeDtypeStruct((NC * NPAD,), jnp.float32)),
        mesh=mesh,
        compiler_params=pltpu.CompilerParams(use_tc_tiling_on_sc=False),
        scratch_types=[
            pltpu.VMEM_SHARED((NPAD, FH), jnp.float32),
            pltpu.VMEM_SHARED((NPAD,), jnp.float32),
            pltpu.VMEM_SHARED((NPAD,), jnp.float32),
            pltpu.VMEM((4, BLKB, CH), jnp.int32),
            pltpu.VMEM((4, BLKB, CH), jnp.int32),
            pltpu.VMEM((BLKB, CH, FH), jnp.float32),
            pltpu.VMEM((BLKB, CH), jnp.float32),
            pltpu.SemaphoreType.DMA,
            pltpu.SemaphoreType.DMA,
            pltpu.SemaphoreType.DMA,
            pltpu.SemaphoreType.DMA,
            pltpu.SemaphoreType.DMA,
        ],
    )


def _prep_body(degA, degB, xr, pA, pB, dinvo):
    i = pl.program_id(0)
    deg = degA[...] + degB[...]
    col = lax.broadcasted_iota(jnp.int32, (BC,), 0) + i * BC
    dinv1 = jnp.where(col < N, lax.rsqrt(deg + 1.0), 0.0)
    dinvo[...] = dinv1
    dcol = dinv1.reshape(BC, 1)
    xb = xr[...]
    pA[...] = jnp.where(dcol > 0, xb[:, :FH] * dcol, 0.0)
    pB[...] = jnp.where(dcol > 0, xb[:, FH:] * dcol, 0.0)


def _prep_call(degp, x):
    return pl.pallas_call(
        _prep_body,
        grid=(NBC,),
        in_specs=[
            pl.BlockSpec((BC,), lambda i: (i,)),
            pl.BlockSpec((BC,), lambda i: (i + NBC,)),
            pl.BlockSpec((BC, F), lambda i: (i, 0)),
        ],
        out_specs=[
            pl.BlockSpec((BC, FH), lambda i: (i, 0)),
            pl.BlockSpec((BC, FH), lambda i: (i, 0)),
            pl.BlockSpec((BC,), lambda i: (i,)),
        ],
        out_shape=[
            jax.ShapeDtypeStruct((NPAD, FH), jnp.float32),
            jax.ShapeDtypeStruct((NPAD, FH), jnp.float32),
            jax.ShapeDtypeStruct((NPAD,), jnp.float32),
        ],
    )(degp, degp, x)


def _final_body(aggA, aggB, pAr, pBr, dinvb, csA, csB,
                W1r, b1c, W2r, b2r, out, acc):
    i = pl.program_id(0)

    @pl.when(i == 0)
    def _():
        acc[...] = jnp.zeros_like(acc)

    dinvrow = dinvb[...].reshape(1, BC)
    cwrow = dinvrow * (csA[...].reshape(1, BC)
                       + csB[...].reshape(1, BC) + dinvrow)
    t = jnp.concatenate([aggA[...] + pAr[...], aggB[...] + pBr[...]],
                        axis=1)
    mT = lax.dot_general(W1r[...], t, (((0,), (1,)), ((), ())),
                         preferred_element_type=jnp.float32)
    hT = jnp.maximum(mT * dinvrow + b1c[...], 0.0)
    acc[...] += jnp.sum(hT * cwrow, axis=1, keepdims=True)

    @pl.when(i == pl.num_programs(0) - 1)
    def _():
        vT = lax.dot_general(W2r[...], acc[...] * (1.0 / N),
                             (((0,), (0,)), ((), ())),
                             preferred_element_type=jnp.float32)
        out[...] = vT.reshape(1, H) + b2r[...]


def _final_call(agg, pA, pB, dinv1d, csp, W1, b1c, W2, b2r):
    return pl.pallas_call(
        _final_body,
        grid=(NBC,),
        in_specs=[
            pl.BlockSpec((BC, FH), lambda i: (i, 0)),
            pl.BlockSpec((BC, FH), lambda i: (i + NBC, 0)),
            pl.BlockSpec((BC, FH), lambda i: (i, 0)),
            pl.BlockSpec((BC, FH), lambda i: (i, 0)),
            pl.BlockSpec((BC,), lambda i: (i,)),
            pl.BlockSpec((BC,), lambda i: (i,)),
            pl.BlockSpec((BC,), lambda i: (i + NBC,)),
            pl.BlockSpec((F, H), lambda i: (0, 0)),
            pl.BlockSpec((H, 1), lambda i: (0, 0)),
            pl.BlockSpec((H, H), lambda i: (0, 0)),
            pl.BlockSpec((1, H), lambda i: (0, 0)),
        ],
        out_specs=pl.BlockSpec((1, H), lambda i: (0, 0)),
        out_shape=jax.ShapeDtypeStruct((1, H), jnp.float32),
        scratch_shapes=[pltpu.VMEM((H, 1), jnp.float32)],
    )(agg, agg, pA, pB, dinv1d, csp, csp, W1, b1c, W2, b2r)


def kernel(x, edge_index, W1, b1, W2, b2):
    ei = edge_index.astype(jnp.int32)
    src2d = ei[0].reshape(ECH, CH)
    dst2d = ei[1].reshape(ECH, CH)

    z1 = jnp.zeros((NPAD,), jnp.float32)
    z16 = jnp.zeros((NPAD, FH), jnp.float32)

    degp = _deg_call()(dst2d, z1)
    pA, pB, dinv1d = _prep_call(degp, x)
    agg, csp = _agg_call()(src2d, dst2d, pA, pB, dinv1d, z16, z1)

    return _final_call(agg, pA, pB, dinv1d, csp,
                       W1, b1.reshape(H, 1), W2, b2.reshape(1, H))
